# baseline (speedup 1.0000x reference)
import jax
import jax.numpy as jnp
import numpy as np
from jax.experimental import pallas as pl

_MIN_CROP_RATE = 0.6
_B, _N = 64, 16384
_rng0 = np.random.default_rng(0)
_CROP1 = float(_rng0.uniform(_MIN_CROP_RATE, 1.0))
_CROP2 = float(_rng0.uniform(_MIN_CROP_RATE, 1.0))
_N1 = int(_N * _CROP1)
_N2 = int(_N * _CROP2)


def _normalize(pts):
    pts_min = jnp.min(pts, axis=1, keepdims=True)
    pts_max = jnp.max(pts, axis=1, keepdims=True)
    center = (pts_min + pts_max) / 2.0
    centered = pts - center
    radius = jnp.max(jnp.linalg.norm(centered, axis=2), axis=1)
    return centered / radius[:, None, None]


def _rotate(pts, key):
    b = pts.shape[0]
    angles = jax.random.uniform(key, (b,), minval=0.0, maxval=2.0 * np.pi)
    c = jnp.cos(angles)
    s = jnp.sin(angles)
    z = jnp.zeros_like(c)
    o = jnp.ones_like(c)
    R = jnp.stack([c, -s, z, s, c, z, z, z, o], axis=1).reshape(b, 3, 3)
    return jnp.einsum('bij,bnj->bni', R, pts)


def _crop(pts, num, key):
    b, n, _ = pts.shape
    center_indices = jax.random.randint(key, (b,), 0, n)
    center_points = jnp.take_along_axis(pts, center_indices[:, None, None], axis=1)
    distances = jnp.linalg.norm(pts - center_points, axis=2)
    _, indices = jax.lax.top_k(-distances, num)
    selected = jnp.take_along_axis(pts, indices[:, :, None], axis=1)
    pts_min = jnp.min(selected, axis=1)
    pts_max = jnp.max(selected, axis=1)
    centers = (pts_min + pts_max) / 2.0
    return _normalize(selected), centers


def _sub_kernel(a_ref, b_ref, o_ref):
    o_ref[...] = a_ref[...] - b_ref[...]


def kernel(pts):
    k = jax.random.key(1)
    k1, k2, kr1, kr2 = jax.random.split(k, 4)
    view1, center1 = _crop(pts, _N1, k1)
    view2, center2 = _crop(pts, _N2, k2)
    view1 = _rotate(view1, kr1)
    view2 = _rotate(view2, kr2)
    relative_center = pl.pallas_call(
        _sub_kernel,
        out_shape=jax.ShapeDtypeStruct(center1.shape, center1.dtype),
    )(center2, center1)
    return (relative_center, view1, view2)


# speedup vs baseline: 1.4348x; 1.4348x over previous
import functools

import jax
import jax.numpy as jnp
import numpy as np
from jax import lax
from jax.experimental import pallas as pl
from jax.experimental.pallas import tpu as pltpu
from jax.experimental.pallas import tpu_sc as plsc

_MIN_CROP_RATE = 0.6
_B, _N = 64, 16384
_rng0 = np.random.default_rng(0)
_CROP1 = float(_rng0.uniform(_MIN_CROP_RATE, 1.0))
_CROP2 = float(_rng0.uniform(_MIN_CROP_RATE, 1.0))
_N1 = int(_N * _CROP1)
_N2 = int(_N * _CROP2)

_L = 16
_CH = _N // _L
_PAD = _N + _L


def _dist_body(pts_ref, c_ref, d_ref):
    b = pl.program_id(0)
    x = pts_ref[0, 0]
    y = pts_ref[0, 1]
    z = pts_ref[0, 2]
    for crop in range(2):
        dx = x - c_ref[crop, b, 0]
        dy = y - c_ref[crop, b, 1]
        dz = z - c_ref[crop, b, 2]
        d_ref[crop, 0] = jnp.sqrt(dx * dx + dy * dy + dz * dz)


def _distances(ptsT4, cpts):
    return pl.pallas_call(
        _dist_body,
        grid=(_B,),
        in_specs=[
            pl.BlockSpec((1, 3, 128, 128), lambda b: (b, 0, 0, 0)),
            pl.BlockSpec((2, _B, 3), lambda b: (0, 0, 0)),
        ],
        out_specs=pl.BlockSpec((2, 1, 128, 128), lambda b: (0, b, 0, 0)),
        out_shape=jax.ShapeDtypeStruct((2, _B, 128, 128), jnp.float32),
    )(ptsT4, cpts)


def _sc_sort_body(d_hbm, idx_hbm, ka, kb, va, vb, hist):
    nc = 2
    wid = lax.axis_index("s") * nc + lax.axis_index("c")
    lanes = lax.broadcasted_iota(jnp.int32, (_L,), 0)
    lane_off = lanes * (_CH + 1)
    ones_i = jnp.ones((_L,), jnp.int32)
    zeros_i = jnp.zeros((_L,), jnp.int32)

    for j in range(2):
        b = wid * 2 + j
        for crop in range(2):
            pltpu.sync_copy(d_hbm.at[crop, b], kb.at[pl.ds(0, _N)])

            def p0(t, _):
                k = kb[pl.ds(t * _L, _L)]
                i = t * _L + lanes
                a = i + (i >> 10)
                plsc.store_scatter(ka, [a], k)
                plsc.store_scatter(va, [a], i)
                return 0

            lax.fori_loop(0, _CH, p0, 0)

            bufs = [(ka, va, kb, vb), (kb, vb, ka, va)]
            for p in range(4):
                sh = 8 * p
                sk, sv, dk, dv = bufs[p % 2]

                def zro(t, _):
                    hist[pl.ds(t * _L, _L)] = zeros_i
                    return 0

                lax.fori_loop(0, 256, zro, 0)

                def hst(t, _):
                    k = plsc.load_gather(sk, [lane_off + t])
                    ki = plsc.bitcast(k, jnp.int32)
                    dg = (ki >> sh) & 255
                    plsc.addupdate_scatter(hist, [dg * _L + lanes], ones_i)
                    return 0

                lax.fori_loop(0, _CH, hst, 0)

                def scn(t, carry):
                    v = hist[pl.ds(t * _L, _L)]
                    inc = plsc.cumsum(v)
                    hist[pl.ds(t * _L, _L)] = inc - v + carry
                    return carry + jnp.max(inc)

                lax.fori_loop(0, 256, scn, jnp.int32(0))

                def sct(t, _):
                    a = lane_off + t
                    k = plsc.load_gather(sk, [a])
                    v = plsc.load_gather(sv, [a])
                    ki = plsc.bitcast(k, jnp.int32)
                    dg = (ki >> sh) & 255
                    hp = dg * _L + lanes
                    o = plsc.load_gather(hist, [hp])
                    plsc.store_scatter(hist, [hp], o + 1)
                    oa = o + (o >> 10)
                    plsc.store_scatter(dk, [oa], k)
                    plsc.store_scatter(dv, [oa], v)
                    return 0

                lax.fori_loop(0, _CH, sct, 0)

            def lin(t, _):
                i = t * _L + lanes
                a = i + (i >> 10)
                v = plsc.load_gather(va, [a])
                vb[pl.ds(t * _L, _L)] = v
                return 0

            lax.fori_loop(0, _CH, lin, 0)
            pltpu.sync_copy(vb.at[pl.ds(0, _N)], idx_hbm.at[crop, b])


def _sc_sort(d):
    mesh = plsc.VectorSubcoreMesh(core_axis_name="c", subcore_axis_name="s")
    f = functools.partial(
        pl.kernel,
        mesh=mesh,
        compiler_params=pltpu.CompilerParams(needs_layout_passes=False),
        out_type=jax.ShapeDtypeStruct((2, _B, _N), jnp.int32),
        scratch_types=[
            pltpu.VMEM((_PAD,), jnp.float32),
            pltpu.VMEM((_PAD,), jnp.float32),
            pltpu.VMEM((_PAD,), jnp.int32),
            pltpu.VMEM((_PAD,), jnp.int32),
            pltpu.VMEM((4096,), jnp.int32),
        ],
    )(_sc_sort_body)
    return f(d)


def _normalize(pts):
    pts_min = jnp.min(pts, axis=1, keepdims=True)
    pts_max = jnp.max(pts, axis=1, keepdims=True)
    center = (pts_min + pts_max) / 2.0
    centered = pts - center
    radius = jnp.max(jnp.linalg.norm(centered, axis=2), axis=1)
    return centered / radius[:, None, None]


def _rotate(pts, angles):
    c = jnp.cos(angles)
    s = jnp.sin(angles)
    z = jnp.zeros_like(c)
    o = jnp.ones_like(c)
    R = jnp.stack([c, -s, z, s, c, z, z, z, o], axis=1).reshape(-1, 3, 3)
    return jnp.einsum('bij,bnj->bni', R, pts)


def _crop_from_idx(pts, indices, num):
    selected = jnp.take_along_axis(pts, indices[:, :num, None], axis=1)
    pts_min = jnp.min(selected, axis=1)
    pts_max = jnp.max(selected, axis=1)
    centers = (pts_min + pts_max) / 2.0
    return _normalize(selected), centers


def kernel(pts):
    k = jax.random.key(1)
    k1, k2, kr1, kr2 = jax.random.split(k, 4)
    ci1 = jax.random.randint(k1, (_B,), 0, _N)
    ci2 = jax.random.randint(k2, (_B,), 0, _N)
    ang1 = jax.random.uniform(kr1, (_B,), minval=0.0, maxval=2.0 * np.pi)
    ang2 = jax.random.uniform(kr2, (_B,), minval=0.0, maxval=2.0 * np.pi)
    c1 = jnp.take_along_axis(pts, ci1[:, None, None], axis=1)[:, 0, :]
    c2 = jnp.take_along_axis(pts, ci2[:, None, None], axis=1)[:, 0, :]
    cpts = jnp.stack([c1, c2], axis=0)
    ptsT4 = pts.transpose(0, 2, 1).reshape(_B, 3, 128, 128)

    d4 = _distances(ptsT4, cpts)
    d = d4.reshape(2, _B, _N)

    idx = _sc_sort(d)

    view1, center1 = _crop_from_idx(pts, idx[0], _N1)
    view2, center2 = _crop_from_idx(pts, idx[1], _N2)
    view1 = _rotate(view1, ang1)
    view2 = _rotate(view2, ang2)
    relative_center = center2 - center1
    return (relative_center, view1, view2)


# speedup vs baseline: 2.1456x; 1.4954x over previous
import functools

import jax
import jax.numpy as jnp
import numpy as np
from jax import lax
from jax.experimental import pallas as pl
from jax.experimental.pallas import tpu as pltpu
from jax.experimental.pallas import tpu_sc as plsc

_MIN_CROP_RATE = 0.6
_B, _N = 64, 16384
_rng0 = np.random.default_rng(0)
_CROP1 = float(_rng0.uniform(_MIN_CROP_RATE, 1.0))
_CROP2 = float(_rng0.uniform(_MIN_CROP_RATE, 1.0))
_N1 = int(_N * _CROP1)
_N2 = int(_N * _CROP2)

_L = 16
_CH = _N // _L
_PAD = _N + _L

_ST = 6144
_KS = (_N1, _N2)
_FULL = (_N1 // 2048, _N2 // 2048)
_TAILP = (_N1 % 2048, _N2 % 2048)
_TAILW = tuple(-(-3 * tp // 128) * 128 for tp in _TAILP)
_OUTW = tuple(f * _ST + w for f, w in zip(_FULL, _TAILW))


def _dist_body(pts_ref, c_ref, d_ref):
    b = pl.program_id(0)
    x = pts_ref[0, 0]
    y = pts_ref[0, 1]
    z = pts_ref[0, 2]
    for crop in range(2):
        dx = x - c_ref[crop, b, 0]
        dy = y - c_ref[crop, b, 1]
        dz = z - c_ref[crop, b, 2]
        d_ref[crop, 0] = jnp.sqrt(dx * dx + dy * dy + dz * dz)


def _distances(ptsT4, cpts):
    return pl.pallas_call(
        _dist_body,
        grid=(_B,),
        in_specs=[
            pl.BlockSpec((1, 3, 128, 128), lambda b: (b, 0, 0, 0)),
            pl.BlockSpec((2, _B, 3), lambda b: (0, 0, 0)),
        ],
        out_specs=pl.BlockSpec((2, 1, 128, 128), lambda b: (0, b, 0, 0)),
        out_shape=jax.ShapeDtypeStruct((2, _B, 128, 128), jnp.float32),
    )(ptsT4, cpts)


def _sc_body(xyz_hbm, d_hbm, trig_hbm, v1_hbm, v2_hbm, rc_hbm,
             xyz, ka, kb, va, vb, hist, stage, trig, rcb):
    nc = 2
    wid = lax.axis_index("s") * nc + lax.axis_index("c")
    lanes = lax.broadcasted_iota(jnp.int32, (_L,), 0)
    lane_off = lanes * (_CH + 1)
    ones_i = jnp.ones((_L,), jnp.int32)
    zeros_i = jnp.zeros((_L,), jnp.int32)
    big = jnp.float32(3.4e38)

    def gxyz(jv):
        a = jv + (jv >> 10)
        p = plsc.load_gather(va, [a])
        x = plsc.load_gather(xyz, [p])
        y = plsc.load_gather(xyz, [p + _N])
        z = plsc.load_gather(xyz, [p + 2 * _N])
        return x, y, z

    for j in range(2):
        b = wid * 2 + j
        for c in range(3):
            pltpu.sync_copy(xyz_hbm.at[b * 3 + c], xyz.at[pl.ds(c * _N, _N)])
        pltpu.sync_copy(trig_hbm.at[b], trig)
        centers = []

        for crop in range(2):
            k_sel = _KS[crop]
            pltpu.sync_copy(d_hbm.at[crop, b], kb.at[pl.ds(0, _N)])

            def p0(t, _):
                kx = kb[pl.ds(t * _L, _L)]
                i = t * _L + lanes
                a = i + (i >> 10)
                plsc.store_scatter(ka, [a], kx)
                plsc.store_scatter(va, [a], i)
                return 0

            lax.fori_loop(0, _CH, p0, 0)

            bufs = [(ka, va, kb, vb), (kb, vb, ka, va)]
            for p in range(4):
                sh = 8 * p
                sk, sv, dk, dv = bufs[p % 2]

                def zro(t, _):
                    hist[pl.ds(t * _L, _L)] = zeros_i
                    return 0

                lax.fori_loop(0, 256, zro, 0)

                def hst(t, _):
                    kx = plsc.load_gather(sk, [lane_off + t])
                    ki = plsc.bitcast(kx, jnp.int32)
                    dg = (ki >> sh) & 255
                    plsc.addupdate_scatter(hist, [dg * _L + lanes], ones_i)
                    return 0

                lax.fori_loop(0, _CH, hst, 0)

                def scn(t, carry):
                    v = hist[pl.ds(t * _L, _L)]
                    inc = plsc.cumsum(v)
                    hist[pl.ds(t * _L, _L)] = inc - v + carry
                    return carry + jnp.max(inc)

                lax.fori_loop(0, 256, scn, jnp.int32(0))

                def sct(t, _):
                    a = lane_off + t
                    kx = plsc.load_gather(sk, [a])
                    v = plsc.load_gather(sv, [a])
                    ki = plsc.bitcast(kx, jnp.int32)
                    dg = (ki >> sh) & 255
                    hp = dg * _L + lanes
                    o = plsc.load_gather(hist, [hp])
                    plsc.store_scatter(hist, [hp], o + 1)
                    oa = o + (o >> 10)
                    plsc.store_scatter(dk, [oa], kx)
                    plsc.store_scatter(dv, [oa], v)
                    return 0

                lax.fori_loop(0, _CH, sct, 0)

            nit = -(-k_sel // _L)

            def mm(t, carry):
                xm, xM, ym, yM, zm, zM = carry
                jv = t * _L + lanes
                x, y, z = gxyz(jv)
                msk = jv < k_sel
                xm = jnp.minimum(xm, jnp.where(msk, x, big))
                xM = jnp.maximum(xM, jnp.where(msk, x, -big))
                ym = jnp.minimum(ym, jnp.where(msk, y, big))
                yM = jnp.maximum(yM, jnp.where(msk, y, -big))
                zm = jnp.minimum(zm, jnp.where(msk, z, big))
                zM = jnp.maximum(zM, jnp.where(msk, z, -big))
                return xm, xM, ym, yM, zm, zM

            full_big = jnp.full((_L,), big, jnp.float32)
            xm, xM, ym, yM, zm, zM = lax.fori_loop(
                0, nit, mm, (full_big, -full_big, full_big, -full_big,
                             full_big, -full_big))
            cx = (jnp.min(xm) + jnp.max(xM)) * 0.5
            cy = (jnp.min(ym) + jnp.max(yM)) * 0.5
            cz = (jnp.min(zm) + jnp.max(zM)) * 0.5
            centers.append((cx, cy, cz))
            cxv = jnp.full((_L,), cx, jnp.float32)
            cyv = jnp.full((_L,), cy, jnp.float32)
            czv = jnp.full((_L,), cz, jnp.float32)

            def rad(t, m):
                jv = t * _L + lanes
                x, y, z = gxyz(jv)
                msk = jv < k_sel
                dx = x - cxv
                dy = y - cyv
                dz = z - czv
                ssq = dx * dx + dy * dy + dz * dz
                return jnp.maximum(m, jnp.where(msk, ssq, 0.0))

            mv = lax.fori_loop(0, nit, rad, jnp.zeros((_L,), jnp.float32))
            m = jnp.max(mv)
            mvec = jnp.full((_L,), m, jnp.float32)
            mi = plsc.bitcast(mvec, jnp.int32)
            r = plsc.bitcast(jnp.int32(0x5F3759DF) - (mi >> 1), jnp.float32)
            for _ in range(4):
                r = r * (1.5 - 0.5 * mvec * r * r)

            tv = trig[...]
            cv = jnp.full((_L,), jnp.sum(jnp.where(lanes == 2 * crop, tv, 0.0)),
                          jnp.float32)
            sv_ = jnp.full((_L,), jnp.sum(jnp.where(lanes == 2 * crop + 1, tv, 0.0)),
                           jnp.float32)
            out_hbm = v1_hbm if crop == 0 else v2_hbm

            def wbody(ci, t, _):
                jv = ci * 2048 + t * _L + lanes
                x, y, z = gxyz(jv)
                xs = (x - cxv) * r
                ys = (y - cyv) * r
                zs = (z - czv) * r
                rx = cv * xs - sv_ * ys
                ry = sv_ * xs + cv * ys
                q = t * 48 + lanes * 3
                plsc.store_scatter(stage, [q], rx)
                plsc.store_scatter(stage, [q + 1], ry)
                plsc.store_scatter(stage, [q + 2], zs)
                return 0

            def wchunk(ci, _):
                lax.fori_loop(0, 128, functools.partial(wbody, ci), 0)
                pltpu.sync_copy(stage,
                                out_hbm.at[b, pl.ds(ci * _ST, _ST)])
                return 0

            fc = _FULL[crop]
            lax.fori_loop(0, fc, wchunk, 0)
            tail_it = -(-_TAILP[crop] // _L)
            lax.fori_loop(0, tail_it, functools.partial(wbody, fc), 0)
            tw = _TAILW[crop]
            pltpu.sync_copy(stage.at[pl.ds(0, tw)],
                            out_hbm.at[b, pl.ds(fc * _ST, tw)])

        (c1x, c1y, c1z), (c2x, c2y, c2z) = centers
        dx = jnp.full((_L,), c2x - c1x, jnp.float32)
        dy = jnp.full((_L,), c2y - c1y, jnp.float32)
        dz = jnp.full((_L,), c2z - c1z, jnp.float32)
        zf = jnp.zeros((_L,), jnp.float32)
        rcb[...] = jnp.where(lanes == 0, dx,
                             jnp.where(lanes == 1, dy,
                                       jnp.where(lanes == 2, dz, zf)))
        pltpu.sync_copy(rcb, rc_hbm.at[b])


def _sc_run(xyzT, d, trig):
    mesh = plsc.VectorSubcoreMesh(core_axis_name="c", subcore_axis_name="s")
    f = functools.partial(
        pl.kernel,
        mesh=mesh,
        compiler_params=pltpu.CompilerParams(needs_layout_passes=False),
        out_type=(
            jax.ShapeDtypeStruct((_B, _OUTW[0]), jnp.float32),
            jax.ShapeDtypeStruct((_B, _OUTW[1]), jnp.float32),
            jax.ShapeDtypeStruct((_B, 16), jnp.float32),
        ),
        scratch_types=[
            pltpu.VMEM((3 * _N,), jnp.float32),
            pltpu.VMEM((_PAD,), jnp.float32),
            pltpu.VMEM((_PAD,), jnp.float32),
            pltpu.VMEM((_PAD,), jnp.int32),
            pltpu.VMEM((_PAD,), jnp.int32),
            pltpu.VMEM((4096,), jnp.int32),
            pltpu.VMEM((_ST,), jnp.float32),
            pltpu.VMEM((_L,), jnp.float32),
            pltpu.VMEM((_L,), jnp.float32),
        ],
    )(_sc_body)
    return f(xyzT, d, trig)


def kernel(pts):
    k = jax.random.key(1)
    k1, k2, kr1, kr2 = jax.random.split(k, 4)
    ci1 = jax.random.randint(k1, (_B,), 0, _N)
    ci2 = jax.random.randint(k2, (_B,), 0, _N)
    ang1 = jax.random.uniform(kr1, (_B,), minval=0.0, maxval=2.0 * np.pi)
    ang2 = jax.random.uniform(kr2, (_B,), minval=0.0, maxval=2.0 * np.pi)
    c1 = jnp.take_along_axis(pts, ci1[:, None, None], axis=1)[:, 0, :]
    c2 = jnp.take_along_axis(pts, ci2[:, None, None], axis=1)[:, 0, :]
    cpts = jnp.stack([c1, c2], axis=0)
    ptsT = pts.transpose(0, 2, 1)
    ptsT4 = ptsT.reshape(_B, 3, 128, 128)

    d4 = _distances(ptsT4, cpts)
    d = d4.reshape(2, _B, _N)

    trig = jnp.stack([jnp.cos(ang1), jnp.sin(ang1),
                      jnp.cos(ang2), jnp.sin(ang2)], axis=1)
    trig = jnp.pad(trig, ((0, 0), (0, 12)))

    v1, v2, rc = _sc_run(ptsT.reshape(_B * 3, _N), d, trig)

    view1 = v1[:, :3 * _N1].reshape(_B, _N1, 3)
    view2 = v2[:, :3 * _N2].reshape(_B, _N2, 3)
    relative_center = rc[:, :3]
    return (relative_center, view1, view2)


# speedup vs baseline: 2.5777x; 1.2014x over previous
import functools

import jax
import jax.numpy as jnp
import numpy as np
from jax import lax
from jax.experimental import pallas as pl
from jax.experimental.pallas import tpu as pltpu
from jax.experimental.pallas import tpu_sc as plsc

_MIN_CROP_RATE = 0.6
_B, _N = 64, 16384
_rng0 = np.random.default_rng(0)
_CROP1 = float(_rng0.uniform(_MIN_CROP_RATE, 1.0))
_CROP2 = float(_rng0.uniform(_MIN_CROP_RATE, 1.0))
_N1 = int(_N * _CROP1)
_N2 = int(_N * _CROP2)

_L = 16
_CH = _N // _L
_PAD = _N + _L

_ST = 6144
_KS = (_N1, _N2)
_FULL = (_N1 // 2048, _N2 // 2048)
_TAILP = (_N1 % 2048, _N2 % 2048)
_TAILW = tuple(-(-3 * tp // 128) * 128 for tp in _TAILP)
_OUTW = tuple(f * _ST + w for f, w in zip(_FULL, _TAILW))


def _dist_body(pts_ref, c_ref, d_ref):
    b = pl.program_id(0)
    x = pts_ref[0, 0]
    y = pts_ref[0, 1]
    z = pts_ref[0, 2]
    for crop in range(2):
        dx = x - c_ref[crop, b, 0]
        dy = y - c_ref[crop, b, 1]
        dz = z - c_ref[crop, b, 2]
        d_ref[crop, 0] = jnp.sqrt(dx * dx + dy * dy + dz * dz)


def _distances(ptsT4, cpts):
    return pl.pallas_call(
        _dist_body,
        grid=(_B,),
        in_specs=[
            pl.BlockSpec((1, 3, 128, 128), lambda b: (b, 0, 0, 0)),
            pl.BlockSpec((2, _B, 3), lambda b: (0, 0, 0)),
        ],
        out_specs=pl.BlockSpec((2, 1, 128, 128), lambda b: (0, b, 0, 0)),
        out_shape=jax.ShapeDtypeStruct((2, _B, 128, 128), jnp.float32),
    )(ptsT4, cpts)


def _sc_body(xyz_hbm, d_hbm, trig_hbm, v1_hbm, v2_hbm, rc_hbm,
             xyz, ka, kb, va, vb, hist, hist2, stage, trig, rcb):
    nc = 2
    wid = lax.axis_index("s") * nc + lax.axis_index("c")
    lanes = lax.broadcasted_iota(jnp.int32, (_L,), 0)
    lane_off = lanes * (_CH + 1)
    ones_i = jnp.ones((_L,), jnp.int32)
    zeros_i = jnp.zeros((_L,), jnp.int32)
    big = jnp.float32(3.4e38)

    for j in range(2):
        b = wid * 2 + j
        for c in range(3):
            pltpu.sync_copy(xyz_hbm.at[b * 3 + c], xyz.at[pl.ds(c * _N, _N)])
        pltpu.sync_copy(trig_hbm.at[b], trig)
        centers = []

        for crop in range(2):
            k_sel = _KS[crop]
            pltpu.sync_copy(d_hbm.at[crop, b], kb.at[pl.ds(0, _N)])

            def zro(t, _):
                hist[pl.ds(t * _L, _L)] = zeros_i
                return 0

            lax.fori_loop(0, 256, zro, 0)

            def p0(t0, _):
                for u in range(4):
                    t = t0 * 4 + u
                    kx = kb[pl.ds(t * _L, _L)]
                    i = t * _L + lanes
                    a = i + (i >> 10)
                    plsc.store_scatter(ka, [a], kx)
                    plsc.store_scatter(va, [a], i)
                    ki = plsc.bitcast(kx, jnp.int32)
                    dg = ki & 255
                    plsc.addupdate_scatter(hist, [dg * _L + (i >> 10)], ones_i)
                return 0

            lax.fori_loop(0, _CH // 4, p0, 0)

            bufs = [(ka, va, kb, vb), (kb, vb, ka, va)]
            hists = [(hist, hist2), (hist2, hist)]
            for p in range(4):
                sh = 8 * p
                sk, sv, dk, dv = bufs[p % 2]
                hc, hn = hists[p % 2]

                def scn(t, carry):
                    v = hc[pl.ds(t * _L, _L)]
                    inc = plsc.cumsum(v)
                    hc[pl.ds(t * _L, _L)] = inc - v + carry
                    if p < 3:
                        hn[pl.ds(t * _L, _L)] = zeros_i
                    return carry + jnp.max(inc)

                lax.fori_loop(0, 256, scn, jnp.int32(0))

                def sct(t0, _):
                    for u in range(2):
                        t = t0 * 2 + u
                        a = lane_off + t
                        kx = plsc.load_gather(sk, [a])
                        v = plsc.load_gather(sv, [a])
                        ki = plsc.bitcast(kx, jnp.int32)
                        dg = (ki >> sh) & 255
                        hp = dg * _L + lanes
                        o = plsc.load_gather(hc, [hp])
                        plsc.store_scatter(hc, [hp], o + 1)
                        oa = o + (o >> 10)
                        plsc.store_scatter(dk, [oa], kx)
                        plsc.store_scatter(dv, [oa], v)
                        if p < 3:
                            dgn = (ki >> (sh + 8)) & 255
                            plsc.addupdate_scatter(
                                hn, [dgn * _L + (o >> 10)], ones_i)
                    return 0

                lax.fori_loop(0, _CH // 2, sct, 0)

            nit = -(-k_sel // _L)

            def mm(t, carry):
                xm, xM, ym, yM, zm, zM = carry
                jv = t * _L + lanes
                a = jv + (jv >> 10)
                pp = plsc.load_gather(va, [a])
                x = plsc.load_gather(xyz, [pp])
                y = plsc.load_gather(xyz, [pp + _N])
                z = plsc.load_gather(xyz, [pp + 2 * _N])
                ka[pl.ds(t * _L, _L)] = x
                kb[pl.ds(t * _L, _L)] = y
                vb[pl.ds(t * _L, _L)] = plsc.bitcast(z, jnp.int32)
                msk = jv < k_sel
                xm = jnp.minimum(xm, jnp.where(msk, x, big))
                xM = jnp.maximum(xM, jnp.where(msk, x, -big))
                ym = jnp.minimum(ym, jnp.where(msk, y, big))
                yM = jnp.maximum(yM, jnp.where(msk, y, -big))
                zm = jnp.minimum(zm, jnp.where(msk, z, big))
                zM = jnp.maximum(zM, jnp.where(msk, z, -big))
                return xm, xM, ym, yM, zm, zM

            full_big = jnp.full((_L,), big, jnp.float32)
            xm, xM, ym, yM, zm, zM = lax.fori_loop(
                0, nit, mm, (full_big, -full_big, full_big, -full_big,
                             full_big, -full_big))
            cx = (jnp.min(xm) + jnp.max(xM)) * 0.5
            cy = (jnp.min(ym) + jnp.max(yM)) * 0.5
            cz = (jnp.min(zm) + jnp.max(zM)) * 0.5
            centers.append((cx, cy, cz))
            cxv = jnp.full((_L,), cx, jnp.float32)
            cyv = jnp.full((_L,), cy, jnp.float32)
            czv = jnp.full((_L,), cz, jnp.float32)

            def rad1(t, m):
                jv = t * _L + lanes
                x = ka[pl.ds(t * _L, _L)]
                y = kb[pl.ds(t * _L, _L)]
                z = plsc.bitcast(vb[pl.ds(t * _L, _L)], jnp.float32)
                msk = jv < k_sel
                dx = x - cxv
                dy = y - cyv
                dz = z - czv
                ssq = dx * dx + dy * dy + dz * dz
                return jnp.maximum(m, jnp.where(msk, ssq, 0.0))

            def rad(t0, m):
                m = rad1(t0 * 2, m)
                return rad1(t0 * 2 + 1, m)

            mv = lax.fori_loop(0, nit // 2, rad, jnp.zeros((_L,), jnp.float32))
            if nit % 2:
                mv = rad1(nit - 1, mv)
            m = jnp.max(mv)
            mvec = jnp.full((_L,), m, jnp.float32)
            mi = plsc.bitcast(mvec, jnp.int32)
            r = plsc.bitcast(jnp.int32(0x5F3759DF) - (mi >> 1), jnp.float32)
            for _ in range(4):
                r = r * (1.5 - 0.5 * mvec * r * r)

            tv = trig[...]
            cv = jnp.full((_L,), jnp.sum(jnp.where(lanes == 2 * crop, tv, 0.0)),
                          jnp.float32)
            sv_ = jnp.full((_L,), jnp.sum(jnp.where(lanes == 2 * crop + 1, tv, 0.0)),
                           jnp.float32)
            out_hbm = v1_hbm if crop == 0 else v2_hbm

            def wbody(ci, t, _):
                g = ci * 128 + t
                x = ka[pl.ds(g * _L, _L)]
                y = kb[pl.ds(g * _L, _L)]
                z = plsc.bitcast(vb[pl.ds(g * _L, _L)], jnp.float32)
                xs = (x - cxv) * r
                ys = (y - cyv) * r
                zs = (z - czv) * r
                rx = cv * xs - sv_ * ys
                ry = sv_ * xs + cv * ys
                q = t * 48 + lanes * 3
                plsc.store_scatter(stage, [q], rx)
                plsc.store_scatter(stage, [q + 1], ry)
                plsc.store_scatter(stage, [q + 2], zs)
                return 0

            def wchunk(ci, _):
                def w2(t0, _):
                    wbody(ci, t0 * 2, 0)
                    wbody(ci, t0 * 2 + 1, 0)
                    return 0

                lax.fori_loop(0, 64, w2, 0)
                pltpu.sync_copy(stage,
                                out_hbm.at[b, pl.ds(ci * _ST, _ST)])
                return 0

            fc = _FULL[crop]
            lax.fori_loop(0, fc, wchunk, 0)
            tail_it = -(-_TAILP[crop] // _L)
            lax.fori_loop(0, tail_it, functools.partial(wbody, fc), 0)
            tw = _TAILW[crop]
            pltpu.sync_copy(stage.at[pl.ds(0, tw)],
                            out_hbm.at[b, pl.ds(fc * _ST, tw)])

        (c1x, c1y, c1z), (c2x, c2y, c2z) = centers
        dx = jnp.full((_L,), c2x - c1x, jnp.float32)
        dy = jnp.full((_L,), c2y - c1y, jnp.float32)
        dz = jnp.full((_L,), c2z - c1z, jnp.float32)
        zf = jnp.zeros((_L,), jnp.float32)
        rcb[...] = jnp.where(lanes == 0, dx,
                             jnp.where(lanes == 1, dy,
                                       jnp.where(lanes == 2, dz, zf)))
        pltpu.sync_copy(rcb, rc_hbm.at[b])


def _sc_run(xyzT, d, trig):
    mesh = plsc.VectorSubcoreMesh(core_axis_name="c", subcore_axis_name="s")
    f = functools.partial(
        pl.kernel,
        mesh=mesh,
        compiler_params=pltpu.CompilerParams(needs_layout_passes=False),
        out_type=(
            jax.ShapeDtypeStruct((_B, _OUTW[0]), jnp.float32),
            jax.ShapeDtypeStruct((_B, _OUTW[1]), jnp.float32),
            jax.ShapeDtypeStruct((_B, 16), jnp.float32),
        ),
        scratch_types=[
            pltpu.VMEM((3 * _N,), jnp.float32),
            pltpu.VMEM((_PAD,), jnp.float32),
            pltpu.VMEM((_PAD,), jnp.float32),
            pltpu.VMEM((_PAD,), jnp.int32),
            pltpu.VMEM((_PAD,), jnp.int32),
            pltpu.VMEM((4096,), jnp.int32),
            pltpu.VMEM((4096,), jnp.int32),
            pltpu.VMEM((_ST,), jnp.float32),
            pltpu.VMEM((_L,), jnp.float32),
            pltpu.VMEM((_L,), jnp.float32),
        ],
    )(_sc_body)
    return f(xyzT, d, trig)


def kernel(pts):
    k = jax.random.key(1)
    k1, k2, kr1, kr2 = jax.random.split(k, 4)
    ci1 = jax.random.randint(k1, (_B,), 0, _N)
    ci2 = jax.random.randint(k2, (_B,), 0, _N)
    ang1 = jax.random.uniform(kr1, (_B,), minval=0.0, maxval=2.0 * np.pi)
    ang2 = jax.random.uniform(kr2, (_B,), minval=0.0, maxval=2.0 * np.pi)
    c1 = jnp.take_along_axis(pts, ci1[:, None, None], axis=1)[:, 0, :]
    c2 = jnp.take_along_axis(pts, ci2[:, None, None], axis=1)[:, 0, :]
    cpts = jnp.stack([c1, c2], axis=0)
    ptsT = pts.transpose(0, 2, 1)
    ptsT4 = ptsT.reshape(_B, 3, 128, 128)

    d4 = _distances(ptsT4, cpts)
    d = d4.reshape(2, _B, _N)

    trig = jnp.stack([jnp.cos(ang1), jnp.sin(ang1),
                      jnp.cos(ang2), jnp.sin(ang2)], axis=1)
    trig = jnp.pad(trig, ((0, 0), (0, 12)))

    v1, v2, rc = _sc_run(ptsT.reshape(_B * 3, _N), d, trig)

    view1 = v1[:, :3 * _N1].reshape(_B, _N1, 3)
    view2 = v2[:, :3 * _N2].reshape(_B, _N2, 3)
    relative_center = rc[:, :3]
    return (relative_center, view1, view2)


# speedup vs baseline: 2.5779x; 1.0001x over previous
import functools

import jax
import jax.numpy as jnp
import numpy as np
from jax import lax
from jax.experimental import pallas as pl
from jax.experimental.pallas import tpu as pltpu
from jax.experimental.pallas import tpu_sc as plsc

_MIN_CROP_RATE = 0.6
_B, _N = 64, 16384
_rng0 = np.random.default_rng(0)
_CROP1 = float(_rng0.uniform(_MIN_CROP_RATE, 1.0))
_CROP2 = float(_rng0.uniform(_MIN_CROP_RATE, 1.0))
_N1 = int(_N * _CROP1)
_N2 = int(_N * _CROP2)

_L = 16
_CH = _N // _L
_PAD = _N + _L

_ST = 6144
_KS = (_N1, _N2)
_FULL = (_N1 // 2048, _N2 // 2048)
_TAILP = (_N1 % 2048, _N2 % 2048)
_TAILW = tuple(-(-3 * tp // 128) * 128 for tp in _TAILP)
_OUTW = tuple(f * _ST + w for f, w in zip(_FULL, _TAILW))


def _dist_body(pts_ref, c_ref, d_ref):
    b = pl.program_id(0)
    x = pts_ref[0, 0]
    y = pts_ref[0, 1]
    z = pts_ref[0, 2]
    for crop in range(2):
        dx = x - c_ref[crop, b, 0]
        dy = y - c_ref[crop, b, 1]
        dz = z - c_ref[crop, b, 2]
        d_ref[crop, 0] = jnp.sqrt(dx * dx + dy * dy + dz * dz)


def _distances(ptsT4, cpts):
    return pl.pallas_call(
        _dist_body,
        grid=(_B,),
        in_specs=[
            pl.BlockSpec((1, 3, 128, 128), lambda b: (b, 0, 0, 0)),
            pl.BlockSpec((2, _B, 3), lambda b: (0, 0, 0)),
        ],
        out_specs=pl.BlockSpec((2, 1, 128, 128), lambda b: (0, b, 0, 0)),
        out_shape=jax.ShapeDtypeStruct((2, _B, 128, 128), jnp.float32),
    )(ptsT4, cpts)


def _sc_body(xyz_hbm, d_hbm, trig_hbm, v1_hbm, v2_hbm, rc_hbm,
             xyz, ka, kb, va, vb, hist, hist2, stage, trig, rcb):
    nc = 2
    wid = lax.axis_index("s") * nc + lax.axis_index("c")
    lanes = lax.broadcasted_iota(jnp.int32, (_L,), 0)
    lane_off = lanes * (_CH + 1)
    ones_i = jnp.ones((_L,), jnp.int32)
    zeros_i = jnp.zeros((_L,), jnp.int32)
    big = jnp.float32(3.4e38)

    for j in range(2):
        b = wid * 2 + j
        for c in range(3):
            pltpu.sync_copy(xyz_hbm.at[b * 3 + c], xyz.at[pl.ds(c * _N, _N)])
        pltpu.sync_copy(trig_hbm.at[b], trig)
        centers = []

        for crop in range(2):
            k_sel = _KS[crop]
            pltpu.sync_copy(d_hbm.at[crop, b], kb.at[pl.ds(0, _N)])

            def zro(t, _):
                hist[pl.ds(t * _L, _L)] = zeros_i
                return 0

            lax.fori_loop(0, 256, zro, 0)

            def p0(t0, _):
                for u in range(4):
                    t = t0 * 4 + u
                    kx = kb[pl.ds(t * _L, _L)]
                    i = t * _L + lanes
                    a = i + (i >> 10)
                    plsc.store_scatter(ka, [a], kx)
                    plsc.store_scatter(va, [a], i)
                    ki = plsc.bitcast(kx, jnp.int32)
                    dg = ki & 255
                    plsc.addupdate_scatter(hist, [dg * _L + (i >> 10)], ones_i)
                return 0

            lax.fori_loop(0, _CH // 4, p0, 0)

            bufs = [(ka, va, kb, vb), (kb, vb, ka, va)]
            hists = [(hist, hist2), (hist2, hist)]
            for p in range(4):
                sh = 8 * p
                sk, sv, dk, dv = bufs[p % 2]
                hc, hn = hists[p % 2]

                def scn(t, carry):
                    v = hc[pl.ds(t * _L, _L)]
                    inc = plsc.cumsum(v)
                    hc[pl.ds(t * _L, _L)] = inc - v + carry
                    if p < 3:
                        hn[pl.ds(t * _L, _L)] = zeros_i
                    return carry + jnp.max(inc)

                lax.fori_loop(0, 256, scn, jnp.int32(0))

                def sct(t0, _):
                    for u in range(2):
                        t = t0 * 2 + u
                        a = lane_off + t
                        kx = plsc.load_gather(sk, [a])
                        v = plsc.load_gather(sv, [a])
                        ki = plsc.bitcast(kx, jnp.int32)
                        dg = (ki >> sh) & 255
                        hp = dg * _L + lanes
                        o = plsc.load_gather(hc, [hp])
                        plsc.store_scatter(hc, [hp], o + 1)
                        oa = o + (o >> 10)
                        plsc.store_scatter(dk, [oa], kx)
                        plsc.store_scatter(dv, [oa], v)
                        if p < 3:
                            dgn = (ki >> (sh + 8)) & 255
                            plsc.addupdate_scatter(
                                hn, [dgn * _L + (o >> 10)], ones_i)
                    return 0

                lax.fori_loop(0, _CH // 2, sct, 0)

            nit = -(-k_sel // _L)

            def mm(t, carry):
                xm, xM, ym, yM, zm, zM = carry
                jv = t * _L + lanes
                a = jv + (jv >> 10)
                pp = plsc.load_gather(va, [a])
                x = plsc.load_gather(xyz, [pp])
                y = plsc.load_gather(xyz, [pp + _N])
                z = plsc.load_gather(xyz, [pp + 2 * _N])
                ka[pl.ds(t * _L, _L)] = x
                kb[pl.ds(t * _L, _L)] = y
                vb[pl.ds(t * _L, _L)] = plsc.bitcast(z, jnp.int32)
                msk = jv < k_sel
                xm = jnp.minimum(xm, jnp.where(msk, x, big))
                xM = jnp.maximum(xM, jnp.where(msk, x, -big))
                ym = jnp.minimum(ym, jnp.where(msk, y, big))
                yM = jnp.maximum(yM, jnp.where(msk, y, -big))
                zm = jnp.minimum(zm, jnp.where(msk, z, big))
                zM = jnp.maximum(zM, jnp.where(msk, z, -big))
                return xm, xM, ym, yM, zm, zM

            full_big = jnp.full((_L,), big, jnp.float32)
            xm, xM, ym, yM, zm, zM = lax.fori_loop(
                0, nit, mm, (full_big, -full_big, full_big, -full_big,
                             full_big, -full_big))
            cx = (jnp.min(xm) + jnp.max(xM)) * 0.5
            cy = (jnp.min(ym) + jnp.max(yM)) * 0.5
            cz = (jnp.min(zm) + jnp.max(zM)) * 0.5
            centers.append((cx, cy, cz))
            cxv = jnp.full((_L,), cx, jnp.float32)
            cyv = jnp.full((_L,), cy, jnp.float32)
            czv = jnp.full((_L,), cz, jnp.float32)

            def rad1(t, m):
                jv = t * _L + lanes
                x = ka[pl.ds(t * _L, _L)]
                y = kb[pl.ds(t * _L, _L)]
                z = plsc.bitcast(vb[pl.ds(t * _L, _L)], jnp.float32)
                msk = jv < k_sel
                dx = x - cxv
                dy = y - cyv
                dz = z - czv
                ssq = dx * dx + dy * dy + dz * dz
                return jnp.maximum(m, jnp.where(msk, ssq, 0.0))

            def rad(t0, m):
                m = rad1(t0 * 2, m)
                return rad1(t0 * 2 + 1, m)

            mv = lax.fori_loop(0, nit // 2, rad, jnp.zeros((_L,), jnp.float32))
            if nit % 2:
                mv = rad1(nit - 1, mv)
            m = jnp.max(mv)
            mvec = jnp.full((_L,), m, jnp.float32)
            mi = plsc.bitcast(mvec, jnp.int32)
            r = plsc.bitcast(jnp.int32(0x5F3759DF) - (mi >> 1), jnp.float32)
            for _ in range(4):
                r = r * (1.5 - 0.5 * mvec * r * r)

            tv = trig[...]
            cv = jnp.full((_L,), jnp.sum(jnp.where(lanes == 2 * crop, tv, 0.0)),
                          jnp.float32)
            sv_ = jnp.full((_L,), jnp.sum(jnp.where(lanes == 2 * crop + 1, tv, 0.0)),
                           jnp.float32)
            out_hbm = v1_hbm if crop == 0 else v2_hbm

            def wbody(ci, t, _):
                g = ci * 128 + t
                x = ka[pl.ds(g * _L, _L)]
                y = kb[pl.ds(g * _L, _L)]
                z = plsc.bitcast(vb[pl.ds(g * _L, _L)], jnp.float32)
                xs = (x - cxv) * r
                ys = (y - cyv) * r
                zs = (z - czv) * r
                rx = cv * xs - sv_ * ys
                ry = sv_ * xs + cv * ys
                q = t * 48 + lanes * 3
                plsc.store_scatter(stage, [q], rx)
                plsc.store_scatter(stage, [q + 1], ry)
                plsc.store_scatter(stage, [q + 2], zs)
                return 0

            def wchunk(ci, _):
                def w2(t0, _):
                    wbody(ci, t0 * 2, 0)
                    wbody(ci, t0 * 2 + 1, 0)
                    return 0

                lax.fori_loop(0, 64, w2, 0)
                pltpu.sync_copy(stage,
                                out_hbm.at[b, pl.ds(ci * _ST, _ST)])
                return 0

            fc = _FULL[crop]
            lax.fori_loop(0, fc, wchunk, 0)
            tail_it = -(-_TAILP[crop] // _L)
            lax.fori_loop(0, tail_it, functools.partial(wbody, fc), 0)
            tw = _TAILW[crop]
            pltpu.sync_copy(stage.at[pl.ds(0, tw)],
                            out_hbm.at[b, pl.ds(fc * _ST, tw)])

        (c1x, c1y, c1z), (c2x, c2y, c2z) = centers
        dx = jnp.full((_L,), c2x - c1x, jnp.float32)
        dy = jnp.full((_L,), c2y - c1y, jnp.float32)
        dz = jnp.full((_L,), c2z - c1z, jnp.float32)
        zf = jnp.zeros((_L,), jnp.float32)
        rcb[...] = jnp.where(lanes == 0, dx,
                             jnp.where(lanes == 1, dy,
                                       jnp.where(lanes == 2, dz, zf)))
        pltpu.sync_copy(rcb, rc_hbm.at[b])


def _sc_run(xyzT, d, trig):
    mesh = plsc.VectorSubcoreMesh(core_axis_name="c", subcore_axis_name="s")
    f = functools.partial(
        pl.kernel,
        mesh=mesh,
        compiler_params=pltpu.CompilerParams(needs_layout_passes=False,
                                             use_tc_tiling_on_sc=True),
        out_type=(
            jax.ShapeDtypeStruct((_B, _OUTW[0]), jnp.float32),
            jax.ShapeDtypeStruct((_B, _OUTW[1]), jnp.float32),
            jax.ShapeDtypeStruct((_B, 16), jnp.float32),
        ),
        scratch_types=[
            pltpu.VMEM((3 * _N,), jnp.float32),
            pltpu.VMEM((_PAD,), jnp.float32),
            pltpu.VMEM((_PAD,), jnp.float32),
            pltpu.VMEM((_PAD,), jnp.int32),
            pltpu.VMEM((_PAD,), jnp.int32),
            pltpu.VMEM((4096,), jnp.int32),
            pltpu.VMEM((4096,), jnp.int32),
            pltpu.VMEM((_ST,), jnp.float32),
            pltpu.VMEM((_L,), jnp.float32),
            pltpu.VMEM((_L,), jnp.float32),
        ],
    )(_sc_body)
    return f(xyzT, d, trig)


def kernel(pts):
    k = jax.random.key(1)
    k1, k2, kr1, kr2 = jax.random.split(k, 4)
    ci1 = jax.random.randint(k1, (_B,), 0, _N)
    ci2 = jax.random.randint(k2, (_B,), 0, _N)
    ang1 = jax.random.uniform(kr1, (_B,), minval=0.0, maxval=2.0 * np.pi)
    ang2 = jax.random.uniform(kr2, (_B,), minval=0.0, maxval=2.0 * np.pi)
    c1 = jnp.take_along_axis(pts, ci1[:, None, None], axis=1)[:, 0, :]
    c2 = jnp.take_along_axis(pts, ci2[:, None, None], axis=1)[:, 0, :]
    cpts = jnp.stack([c1, c2], axis=0)
    ptsT = pts.transpose(0, 2, 1)
    ptsT4 = ptsT.reshape(_B, 3, 128, 128)

    d4 = _distances(ptsT4, cpts)
    d = d4.reshape(2, _B, _N)

    trig = jnp.stack([jnp.cos(ang1), jnp.sin(ang1),
                      jnp.cos(ang2), jnp.sin(ang2)], axis=1)
    trig = jnp.pad(trig, ((0, 0), (0, 12)))

    v1, v2, rc = _sc_run(ptsT.reshape(_B * 3, _N), d, trig)

    view1 = v1[:, :3 * _N1].reshape(_B, _N1, 3)
    view2 = v2[:, :3 * _N2].reshape(_B, _N2, 3)
    relative_center = rc[:, :3]
    return (relative_center, view1, view2)


# speedup vs baseline: 2.6558x; 1.0302x over previous
import functools

import jax
import jax.numpy as jnp
import numpy as np
from jax import lax
from jax.experimental import pallas as pl
from jax.experimental.pallas import tpu as pltpu
from jax.experimental.pallas import tpu_sc as plsc

_MIN_CROP_RATE = 0.6
_B, _N = 64, 16384
_rng0 = np.random.default_rng(0)
_CROP1 = float(_rng0.uniform(_MIN_CROP_RATE, 1.0))
_CROP2 = float(_rng0.uniform(_MIN_CROP_RATE, 1.0))
_N1 = int(_N * _CROP1)
_N2 = int(_N * _CROP2)

_L = 16
_CH = _N // _L
_PAD = _N + _L

_ST = 6144
_KS = (_N1, _N2)
_FULL = (_N1 // 2048, _N2 // 2048)
_TAILP = (_N1 % 2048, _N2 % 2048)
_TAILW = tuple(-(-3 * tp // 128) * 128 for tp in _TAILP)
_OUTW = tuple(f * _ST + w for f, w in zip(_FULL, _TAILW))


def _dist_body(pts_ref, c_ref, d_ref):
    b = pl.program_id(0)
    x = pts_ref[0, 0]
    y = pts_ref[0, 1]
    z = pts_ref[0, 2]
    for crop in range(2):
        dx = x - c_ref[crop, b, 0]
        dy = y - c_ref[crop, b, 1]
        dz = z - c_ref[crop, b, 2]
        d_ref[crop, 0] = jnp.sqrt(dx * dx + dy * dy + dz * dz)


def _distances(ptsT4, cpts):
    return pl.pallas_call(
        _dist_body,
        grid=(_B,),
        in_specs=[
            pl.BlockSpec((1, 3, 128, 128), lambda b: (b, 0, 0, 0)),
            pl.BlockSpec((2, _B, 3), lambda b: (0, 0, 0)),
        ],
        out_specs=pl.BlockSpec((2, 1, 128, 128), lambda b: (0, b, 0, 0)),
        out_shape=jax.ShapeDtypeStruct((2, _B, 128, 128), jnp.float32),
    )(ptsT4, cpts)


def _sc_body(xyz_hbm, d_hbm, trig_hbm, v1_hbm, v2_hbm, rc_hbm,
             xyz, ka, kb, va, vb, hist, hist2, stage, trig, rcb):
    nc = 2
    wid = lax.axis_index("s") * nc + lax.axis_index("c")
    lanes = lax.broadcasted_iota(jnp.int32, (_L,), 0)
    lane_off = lanes * (_CH + 1)
    ones_i = jnp.ones((_L,), jnp.int32)
    zeros_i = jnp.zeros((_L,), jnp.int32)
    big = jnp.float32(3.4e38)

    for j in range(2):
        b = wid * 2 + j
        for c in range(3):
            pltpu.sync_copy(xyz_hbm.at[b * 3 + c], xyz.at[pl.ds(c * _N, _N)])
        pltpu.sync_copy(trig_hbm.at[b], trig)
        centers = []

        for crop in range(2):
            k_sel = _KS[crop]
            pltpu.sync_copy(d_hbm.at[crop, b], kb.at[pl.ds(0, _N)])

            def zro(t, _):
                hist[pl.ds(t * _L, _L)] = zeros_i
                return 0

            lax.fori_loop(0, 256, zro, 0)

            def p0(t0, _):
                for u in range(4):
                    t = t0 * 4 + u
                    kx = kb[pl.ds(t * _L, _L)]
                    i = t * _L + lanes
                    a = i + (i >> 10)
                    plsc.store_scatter(ka, [a], kx)
                    plsc.store_scatter(va, [a], i)
                    ki = plsc.bitcast(kx, jnp.int32)
                    dg = ki & 255
                    plsc.addupdate_scatter(hist, [dg * _L + (i >> 10)], ones_i)
                return 0

            lax.fori_loop(0, _CH // 4, p0, 0)

            bufs = [(ka, va, kb, vb), (kb, vb, ka, va)]
            hists = [(hist, hist2), (hist2, hist)]
            for p in range(4):
                sh = 8 * p
                sk, sv, dk, dv = bufs[p % 2]
                hc, hn = hists[p % 2]

                def scn(t, carry):
                    v = hc[pl.ds(t * _L, _L)]
                    inc = plsc.cumsum(v)
                    hc[pl.ds(t * _L, _L)] = inc - v + carry
                    if p < 3:
                        hn[pl.ds(t * _L, _L)] = zeros_i
                    return carry + jnp.max(inc)

                lax.fori_loop(0, 256, scn, jnp.int32(0))

                def sct(t0, _):
                    for u in range(4):
                        t = t0 * 4 + u
                        a = lane_off + t
                        kx = plsc.load_gather(sk, [a])
                        v = plsc.load_gather(sv, [a])
                        ki = plsc.bitcast(kx, jnp.int32)
                        dg = (ki >> sh) & 255
                        hp = dg * _L + lanes
                        o = plsc.load_gather(hc, [hp])
                        plsc.store_scatter(hc, [hp], o + 1)
                        if p < 3:
                            oa = o + (o >> 10)
                            plsc.store_scatter(dk, [oa], kx)
                            plsc.store_scatter(dv, [oa], v)
                            dgn = (ki >> (sh + 8)) & 255
                            plsc.addupdate_scatter(
                                hn, [dgn * _L + (o >> 10)], ones_i)
                        else:
                            plsc.store_scatter(dv, [o], v)
                    return 0

                lax.fori_loop(0, _CH // 4, sct, 0)

            nit = -(-k_sel // _L)

            def mm(t, carry):
                xm, xM, ym, yM, zm, zM = carry
                jv = t * _L + lanes
                pp = va[pl.ds(t * _L, _L)]
                x = plsc.load_gather(xyz, [pp])
                y = plsc.load_gather(xyz, [pp + _N])
                z = plsc.load_gather(xyz, [pp + 2 * _N])
                ka[pl.ds(t * _L, _L)] = x
                kb[pl.ds(t * _L, _L)] = y
                vb[pl.ds(t * _L, _L)] = plsc.bitcast(z, jnp.int32)
                msk = jv < k_sel
                xm = jnp.minimum(xm, jnp.where(msk, x, big))
                xM = jnp.maximum(xM, jnp.where(msk, x, -big))
                ym = jnp.minimum(ym, jnp.where(msk, y, big))
                yM = jnp.maximum(yM, jnp.where(msk, y, -big))
                zm = jnp.minimum(zm, jnp.where(msk, z, big))
                zM = jnp.maximum(zM, jnp.where(msk, z, -big))
                return xm, xM, ym, yM, zm, zM

            full_big = jnp.full((_L,), big, jnp.float32)
            xm, xM, ym, yM, zm, zM = lax.fori_loop(
                0, nit, mm, (full_big, -full_big, full_big, -full_big,
                             full_big, -full_big))
            cx = (jnp.min(xm) + jnp.max(xM)) * 0.5
            cy = (jnp.min(ym) + jnp.max(yM)) * 0.5
            cz = (jnp.min(zm) + jnp.max(zM)) * 0.5
            centers.append((cx, cy, cz))
            cxv = jnp.full((_L,), cx, jnp.float32)
            cyv = jnp.full((_L,), cy, jnp.float32)
            czv = jnp.full((_L,), cz, jnp.float32)

            def rad1(t, m):
                jv = t * _L + lanes
                x = ka[pl.ds(t * _L, _L)]
                y = kb[pl.ds(t * _L, _L)]
                z = plsc.bitcast(vb[pl.ds(t * _L, _L)], jnp.float32)
                msk = jv < k_sel
                dx = x - cxv
                dy = y - cyv
                dz = z - czv
                ssq = dx * dx + dy * dy + dz * dz
                return jnp.maximum(m, jnp.where(msk, ssq, 0.0))

            def rad(t0, m):
                m = rad1(t0 * 2, m)
                return rad1(t0 * 2 + 1, m)

            mv = lax.fori_loop(0, nit // 2, rad, jnp.zeros((_L,), jnp.float32))
            if nit % 2:
                mv = rad1(nit - 1, mv)
            m = jnp.max(mv)
            mvec = jnp.full((_L,), m, jnp.float32)
            mi = plsc.bitcast(mvec, jnp.int32)
            r = plsc.bitcast(jnp.int32(0x5F3759DF) - (mi >> 1), jnp.float32)
            for _ in range(4):
                r = r * (1.5 - 0.5 * mvec * r * r)

            tv = trig[...]
            cv = jnp.full((_L,), jnp.sum(jnp.where(lanes == 2 * crop, tv, 0.0)),
                          jnp.float32)
            sv_ = jnp.full((_L,), jnp.sum(jnp.where(lanes == 2 * crop + 1, tv, 0.0)),
                           jnp.float32)
            out_hbm = v1_hbm if crop == 0 else v2_hbm

            def wbody(ci, t, _):
                g = ci * 128 + t
                x = ka[pl.ds(g * _L, _L)]
                y = kb[pl.ds(g * _L, _L)]
                z = plsc.bitcast(vb[pl.ds(g * _L, _L)], jnp.float32)
                xs = (x - cxv) * r
                ys = (y - cyv) * r
                zs = (z - czv) * r
                rx = cv * xs - sv_ * ys
                ry = sv_ * xs + cv * ys
                q = t * 48 + lanes * 3
                plsc.store_scatter(stage, [q], rx)
                plsc.store_scatter(stage, [q + 1], ry)
                plsc.store_scatter(stage, [q + 2], zs)
                return 0

            def wchunk(ci, _):
                def w2(t0, _):
                    wbody(ci, t0 * 2, 0)
                    wbody(ci, t0 * 2 + 1, 0)
                    return 0

                lax.fori_loop(0, 64, w2, 0)
                pltpu.sync_copy(stage,
                                out_hbm.at[b, pl.ds(ci * _ST, _ST)])
                return 0

            fc = _FULL[crop]
            lax.fori_loop(0, fc, wchunk, 0)
            tail_it = -(-_TAILP[crop] // _L)
            lax.fori_loop(0, tail_it, functools.partial(wbody, fc), 0)
            tw = _TAILW[crop]
            pltpu.sync_copy(stage.at[pl.ds(0, tw)],
                            out_hbm.at[b, pl.ds(fc * _ST, tw)])

        (c1x, c1y, c1z), (c2x, c2y, c2z) = centers
        dx = jnp.full((_L,), c2x - c1x, jnp.float32)
        dy = jnp.full((_L,), c2y - c1y, jnp.float32)
        dz = jnp.full((_L,), c2z - c1z, jnp.float32)
        zf = jnp.zeros((_L,), jnp.float32)
        rcb[...] = jnp.where(lanes == 0, dx,
                             jnp.where(lanes == 1, dy,
                                       jnp.where(lanes == 2, dz, zf)))
        pltpu.sync_copy(rcb, rc_hbm.at[b])


def _sc_run(xyzT, d, trig):
    mesh = plsc.VectorSubcoreMesh(core_axis_name="c", subcore_axis_name="s")
    f = functools.partial(
        pl.kernel,
        mesh=mesh,
        compiler_params=pltpu.CompilerParams(needs_layout_passes=False),
        out_type=(
            jax.ShapeDtypeStruct((_B, _OUTW[0]), jnp.float32),
            jax.ShapeDtypeStruct((_B, _OUTW[1]), jnp.float32),
            jax.ShapeDtypeStruct((_B, 16), jnp.float32),
        ),
        scratch_types=[
            pltpu.VMEM((3 * _N,), jnp.float32),
            pltpu.VMEM((_PAD,), jnp.float32),
            pltpu.VMEM((_PAD,), jnp.float32),
            pltpu.VMEM((_PAD,), jnp.int32),
            pltpu.VMEM((_PAD,), jnp.int32),
            pltpu.VMEM((4096,), jnp.int32),
            pltpu.VMEM((4096,), jnp.int32),
            pltpu.VMEM((_ST,), jnp.float32),
            pltpu.VMEM((_L,), jnp.float32),
            pltpu.VMEM((_L,), jnp.float32),
        ],
    )(_sc_body)
    return f(xyzT, d, trig)


def kernel(pts):
    k = jax.random.key(1)
    k1, k2, kr1, kr2 = jax.random.split(k, 4)
    ci1 = jax.random.randint(k1, (_B,), 0, _N)
    ci2 = jax.random.randint(k2, (_B,), 0, _N)
    ang1 = jax.random.uniform(kr1, (_B,), minval=0.0, maxval=2.0 * np.pi)
    ang2 = jax.random.uniform(kr2, (_B,), minval=0.0, maxval=2.0 * np.pi)
    c1 = jnp.take_along_axis(pts, ci1[:, None, None], axis=1)[:, 0, :]
    c2 = jnp.take_along_axis(pts, ci2[:, None, None], axis=1)[:, 0, :]
    cpts = jnp.stack([c1, c2], axis=0)
    ptsT = pts.transpose(0, 2, 1)
    ptsT4 = ptsT.reshape(_B, 3, 128, 128)

    d4 = _distances(ptsT4, cpts)
    d = d4.reshape(2, _B, _N)

    trig = jnp.stack([jnp.cos(ang1), jnp.sin(ang1),
                      jnp.cos(ang2), jnp.sin(ang2)], axis=1)
    trig = jnp.pad(trig, ((0, 0), (0, 12)))

    v1, v2, rc = _sc_run(ptsT.reshape(_B * 3, _N), d, trig)

    view1 = v1[:, :3 * _N1].reshape(_B, _N1, 3)
    view2 = v2[:, :3 * _N2].reshape(_B, _N2, 3)
    relative_center = rc[:, :3]
    return (relative_center, view1, view2)
